# baseline (speedup 1.0000x reference)
import functools

import jax
import jax.numpy as jnp
from jax import lax
from jax.experimental import pallas as pl
from jax.experimental.pallas import tpu as pltpu
from jax.experimental.pallas import tpu_sc as plsc

D_MODEL = 1024
SCALE = 32.0
NC = 2
NS = 16
NW = NC * NS
LANES = 16
C = 32


@functools.partial(jax.jit, static_argnums=(2,))
def _emb(idx, table, B):
    chunks = B // (NW * C)
    mesh = plsc.VectorSubcoreMesh(core_axis_name="c", subcore_axis_name="s")

    @functools.partial(
        pl.kernel,
        out_type=jax.ShapeDtypeStruct((B, D_MODEL), jnp.float32),
        mesh=mesh,
        scratch_types=[
            pltpu.VMEM((chunks, C), jnp.int32),
            pltpu.VMEM((C, D_MODEL), jnp.float32),
            pltpu.VMEM((C, D_MODEL), jnp.float32),
            pltpu.VMEM((C, D_MODEL), jnp.float32),
            pltpu.SemaphoreType.DMA,
            pltpu.SemaphoreType.DMA,
            pltpu.SemaphoreType.DMA,
        ],
    )
    def emb_kernel(idx_hbm, table_hbm, out_hbm, idx_v,
                   in0, in1, outb, si0, si1, so):
        wid = lax.axis_index("s") * NC + lax.axis_index("c")
        base = wid * (chunks * C)
        pltpu.sync_copy(idx_hbm.at[wid], idx_v)
        pltpu.async_copy(table_hbm.at[idx_v.at[0]], in0, si0)
        pltpu.async_copy(table_hbm.at[idx_v.at[1]], in1, si1)
        bufs = ((in0, si0), (in1, si1))

        def outer(jj, carry):
            for b, (inb, sib) in enumerate(bufs):
                j = 2 * jj + b
                pltpu.make_async_copy(table_hbm.at[idx_v.at[j]], inb, sib).wait()

                @pl.when(j >= 1)
                def _():
                    pltpu.make_async_copy(
                        outb, out_hbm.at[pl.ds(base, C)], so).wait()

                @plsc.parallel_loop(0, C, 1)
                def row_body(r):
                    for k in range(D_MODEL // LANES):
                        sl = pl.ds(k * LANES, LANES)
                        outb[r, sl] = inb[r, sl] * SCALE

                @pl.when(j < chunks - 2)
                def _():
                    pltpu.async_copy(table_hbm.at[idx_v.at[j + 2]], inb, sib)

                pltpu.async_copy(outb, out_hbm.at[pl.ds(base + j * C, C)], so)
            return carry

        lax.fori_loop(0, chunks // 2, outer, 0)
        pltpu.make_async_copy(
            outb, out_hbm.at[pl.ds(base + (chunks - 1) * C, C)], so).wait()

    return emb_kernel(idx, table)


def kernel(x, table):
    b, s = x.shape
    B = b * s
    idx = x.reshape(NW, B // (NW * C), C).astype(jnp.int32)
    out = _emb(idx, table, B)
    return out.reshape(b, s, D_MODEL)
